# baseline (speedup 1.0000x reference)
import functools

import jax
import jax.numpy as jnp
from jax import lax
from jax.experimental import pallas as pl
from jax.experimental.pallas import tpu as pltpu
from jax.experimental.pallas import tpu_sc as plsc

_INFO = plsc.get_sparse_core_info()
_NC = _INFO.num_cores
_NS = _INFO.num_subcores
_NW = _NC * _NS

_B = 4096 * 200
_D = 64
_PER_W = _B // _NW
_C = 128
_CHUNKS = _PER_W // _C


def _make_gather():
  mesh = plsc.VectorSubcoreMesh(core_axis_name="c", subcore_axis_name="s")

  @functools.partial(
      pl.kernel,
      mesh=mesh,
      out_type=jax.ShapeDtypeStruct((_B, _D), jnp.float32),
      scratch_types=[
          pltpu.VMEM((_PER_W,), jnp.int32),
          pltpu.VMEM((_C, _D), jnp.float32),
          pltpu.SemaphoreType.DMA,
      ],
      compiler_params=pltpu.CompilerParams(use_tc_tiling_on_sc=False),
  )
  def gather_kernel(idx_hbm, table_hbm, out_hbm, idx_v, rows_v, sem):
    wid = lax.axis_index("s") * _NC + lax.axis_index("c")
    base = wid * _PER_W
    pltpu.sync_copy(idx_hbm.at[pl.ds(base, _PER_W)], idx_v)

    @pl.loop(0, _CHUNKS)
    def _chunk(j):
      off = j * _C
      pltpu.async_copy(
          table_hbm.at[idx_v.at[pl.ds(off, _C)]], rows_v, sem
      ).wait()
      pltpu.sync_copy(rows_v, out_hbm.at[pl.ds(base + off, _C)])

  return gather_kernel


_gather = _make_gather()


@jax.jit
def kernel(X, tok_emb):
  flat_idx = X.reshape(-1).astype(jnp.int32)
  out = _gather(flat_idx, tok_emb)
  return out.reshape(X.shape[0], X.shape[1], _D)


# speedup vs baseline: 1.1112x; 1.1112x over previous
import functools

import jax
import jax.numpy as jnp
from jax import lax
from jax.experimental import pallas as pl
from jax.experimental.pallas import tpu as pltpu
from jax.experimental.pallas import tpu_sc as plsc

_INFO = plsc.get_sparse_core_info()
_NC = _INFO.num_cores
_NS = _INFO.num_subcores
_NW = _NC * _NS

_B = 4096 * 200
_D = 64
_PER_W = _B // _NW
_C = 128
_SC = 512
_NSUB = _SC // _C
_NBUF = 2
_SCHUNKS = _PER_W // _SC


def _make_gather():
  mesh = plsc.VectorSubcoreMesh(core_axis_name="c", subcore_axis_name="s")

  @functools.partial(
      pl.kernel,
      mesh=mesh,
      out_type=jax.ShapeDtypeStruct((_B, _D), jnp.float32),
      scratch_types=[
          pltpu.VMEM((_PER_W,), jnp.int32),
          pltpu.VMEM((_SC, _D), jnp.float32),
          pltpu.VMEM((_SC, _D), jnp.float32),
          pltpu.SemaphoreType.DMA,
          pltpu.SemaphoreType.DMA,
          pltpu.SemaphoreType.DMA,
      ],
      compiler_params=pltpu.CompilerParams(use_tc_tiling_on_sc=False),
  )
  def gather_kernel(idx_hbm, table_hbm, out_hbm, idx_v, rows0, rows1,
                    gsem, ssem0, ssem1):
    wid = lax.axis_index("s") * _NC + lax.axis_index("c")
    base = wid * _PER_W
    pltpu.sync_copy(idx_hbm.at[pl.ds(base, _PER_W)], idx_v)
    rows = [rows0, rows1]
    ssem = [ssem0, ssem1]

    @pl.loop(0, _SCHUNKS, step=_NBUF)
    def _outer(j0):
      for b in range(_NBUF):
        j = j0 + b
        off = j * _SC

        @pl.when(j >= _NBUF)
        def _drain_store():
          pltpu.make_async_copy(
              rows[b], out_hbm.at[pl.ds(base, _SC)], ssem[b]
          ).wait()

        for s in range(_NSUB):
          pltpu.async_copy(
              table_hbm.at[idx_v.at[pl.ds(off + s * _C, _C)]],
              rows[b].at[pl.ds(s * _C, _C)],
              gsem,
          )
        for s in range(_NSUB):
          pltpu.make_async_copy(
              table_hbm.at[idx_v.at[pl.ds(off, _C)]],
              rows[b].at[pl.ds(0, _C)],
              gsem,
          ).wait()
        pltpu.async_copy(rows[b], out_hbm.at[pl.ds(base + off, _SC)], ssem[b])

    for b in range(_NBUF):
      pltpu.make_async_copy(
          rows[b], out_hbm.at[pl.ds(base, _SC)], ssem[b]
      ).wait()

  return gather_kernel


_gather = _make_gather()


@jax.jit
def kernel(X, tok_emb):
  flat_idx = X.reshape(-1).astype(jnp.int32)
  out = _gather(flat_idx, tok_emb)
  return out.reshape(X.shape[0], X.shape[1], _D)
